# baseline (speedup 1.0000x reference)
import jax
import jax.numpy as jnp
from jax.experimental import pallas as pl
from jax.experimental.pallas import tpu as pltpu

N_REAL = 2049
NP = 2304
THRESH2 = 0.04 * 0.04
BI = 768
NI = NP // BI
D = 128


def _split(v):
    hi = v.astype(jnp.bfloat16)
    lo = (v - hi.astype(jnp.float32)).astype(jnp.bfloat16)
    return hi, lo


def _adj_body(xc_ref, yc_ref, xr_ref, yr_ref, a_ref):
    xi = xc_ref[:, 0:1]
    yi = yc_ref[:, 0:1]
    xj = xr_ref[0:1, :]
    yj = yr_ref[0:1, :]
    dx = xi - xj
    dy = yi - yj
    d2 = dx * dx + dy * dy
    a_ref[...] = (d2 <= THRESH2).astype(jnp.bfloat16)


def _fv1_body(a_ref, xc_ref, yc_ref, dc_ref, w_ref, b_ref, out_ref):
    fv0 = jnp.maximum(
        xc_ref[:, 0:1] * w_ref[0:1, :]
        + yc_ref[:, 0:1] * w_ref[1:2, :]
        + dc_ref[:, 0:1] * w_ref[2:3, :]
        + b_ref[0:1, :], 0.0)
    fhi, flo = _split(fv0)
    a = a_ref[...]
    out_ref[...] = (
        jax.lax.dot(a, fhi, preferred_element_type=jnp.float32)
        + jax.lax.dot(a, flo, preferred_element_type=jnp.float32))


def _m_body(ai_ref, a_ref, m_ref):
    cnt = jax.lax.dot(ai_ref[...], a_ref[...],
                      preferred_element_type=jnp.float32)
    m_ref[...] = (cnt > 0.5).astype(jnp.bfloat16)


def _fv2_body(mi_ref, a_ref, fv1_ref, out_ref):
    mi = mi_ref[...]
    ma = jax.lax.dot(mi, a_ref[...], preferred_element_type=jnp.float32)
    c = ma * mi.astype(jnp.float32)
    chi, clo = _split(c)
    f1hi, f1lo = _split(fv1_ref[...])
    out_ref[...] = (
        jax.lax.dot(chi, f1hi, preferred_element_type=jnp.float32)
        + jax.lax.dot(chi, f1lo, preferred_element_type=jnp.float32)
        + jax.lax.dot(clo, f1hi, preferred_element_type=jnp.float32))


_adj = pl.pallas_call(
    _adj_body,
    grid=(NI, NI),
    in_specs=[
        pl.BlockSpec((BI, 128), lambda i, j: (i, 0)),
        pl.BlockSpec((BI, 128), lambda i, j: (i, 0)),
        pl.BlockSpec((8, BI), lambda i, j: (0, j)),
        pl.BlockSpec((8, BI), lambda i, j: (0, j)),
    ],
    out_specs=pl.BlockSpec((BI, BI), lambda i, j: (i, j)),
    out_shape=jax.ShapeDtypeStruct((NP, NP), jnp.bfloat16),
)

_fv1 = pl.pallas_call(
    _fv1_body,
    grid=(1,),
    in_specs=[
        pl.BlockSpec((NP, NP), lambda i: (0, 0)),
        pl.BlockSpec((NP, 128), lambda i: (0, 0)),
        pl.BlockSpec((NP, 128), lambda i: (0, 0)),
        pl.BlockSpec((NP, 128), lambda i: (0, 0)),
        pl.BlockSpec((8, 128), lambda i: (0, 0)),
        pl.BlockSpec((8, 128), lambda i: (0, 0)),
    ],
    out_specs=pl.BlockSpec((NP, D), lambda i: (0, 0)),
    out_shape=jax.ShapeDtypeStruct((NP, D), jnp.float32),
)

_m = pl.pallas_call(
    _m_body,
    grid=(NI,),
    in_specs=[
        pl.BlockSpec((BI, NP), lambda i: (i, 0)),
        pl.BlockSpec((NP, NP), lambda i: (0, 0)),
    ],
    out_specs=pl.BlockSpec((BI, NP), lambda i: (i, 0)),
    out_shape=jax.ShapeDtypeStruct((NP, NP), jnp.bfloat16),
)

_fv2 = pl.pallas_call(
    _fv2_body,
    grid=(NI,),
    in_specs=[
        pl.BlockSpec((BI, NP), lambda i: (i, 0)),
        pl.BlockSpec((NP, NP), lambda i: (0, 0)),
        pl.BlockSpec((NP, D), lambda i: (0, 0)),
    ],
    out_specs=pl.BlockSpec((BI, D), lambda i: (i, 0)),
    out_shape=jax.ShapeDtypeStruct((NP, D), jnp.float32),
)


def kernel(node_locations, time_deadline, W0, b0):
    depot = jax.random.uniform(jax.random.key(1), (1, 2), dtype=jnp.float32)
    loc = jnp.concatenate([depot, node_locations], axis=0)
    tdc = jnp.concatenate(
        [jnp.zeros((1,), jnp.float32), time_deadline[:, 0]], axis=0)
    pad = NP - N_REAL
    x = jnp.concatenate([loc[:, 0], jnp.full((pad,), 1000.0, jnp.float32)])
    y = jnp.concatenate([loc[:, 1], jnp.full((pad,), 2000.0, jnp.float32)])
    t = jnp.concatenate([tdc, jnp.zeros((pad,), jnp.float32)])

    xc = jnp.broadcast_to(x[:, None], (NP, 128))
    yc = jnp.broadcast_to(y[:, None], (NP, 128))
    dc = jnp.broadcast_to(t[:, None], (NP, 128))
    xr = jnp.broadcast_to(x[None, :], (8, NP))
    yr = jnp.broadcast_to(y[None, :], (8, NP))

    wpad = jnp.zeros((8, 128), jnp.float32).at[0:3, :].set(W0.T)
    bpad = jnp.zeros((8, 128), jnp.float32).at[0, :].set(b0)

    a = _adj(xc, yc, xr, yr)
    fv1 = _fv1(a, xc, yc, dc, wpad, bpad)
    m = _m(a, a)
    fv2 = _fv2(m, a, fv1)
    return fv2[:N_REAL]


# speedup vs baseline: 1.3688x; 1.3688x over previous
import jax
import jax.numpy as jnp
from jax.experimental import pallas as pl
from jax.experimental.pallas import tpu as pltpu

N_REAL = 2049
NP = 2304
THRESH2 = 0.04 * 0.04
BI = 768
NI = NP // BI
D = 128
F8 = jnp.float8_e4m3fn


def _adj_body(xc_ref, yc_ref, xr_ref, yr_ref, a_ref):
    xi = xc_ref[:, 0:1]
    yi = yc_ref[:, 0:1]
    xj = xr_ref[0:1, :]
    yj = yr_ref[0:1, :]
    dx = xi - xj
    dy = yi - yj
    d2 = dx * dx + dy * dy
    a_ref[...] = (d2 <= THRESH2).astype(F8)


def _fv1_body(a_ref, xc_ref, yc_ref, tc_ref, w_ref, b_ref, out_ref):
    xk = xc_ref[:, 0:1]
    yk = yc_ref[:, 0:1]
    tk = tc_ref[:, 0:1]
    wx = w_ref[0:1, :]
    wy = w_ref[1:2, :]
    wt = w_ref[2:3, :]
    bb = b_ref[0:1, :]
    fv0 = jnp.maximum(xk * wx + yk * wy + tk * wt + bb, 0.0)
    hi = fv0.astype(jnp.bfloat16)
    lo = (fv0 - hi.astype(jnp.float32)).astype(jnp.bfloat16)
    a = a_ref[...].astype(jnp.bfloat16)
    out_ref[...] = (
        jax.lax.dot(a, hi, preferred_element_type=jnp.float32)
        + jax.lax.dot(a, lo, preferred_element_type=jnp.float32))


def _m_body(a1_ref, a2_ref, m_ref):
    cnt = jax.lax.dot(a1_ref[...], a2_ref[...],
                      preferred_element_type=jnp.float32)
    m_ref[...] = (cnt > 0.5).astype(F8)


def _fv2_body(m1_ref, a2_ref, mij_ref, fv1_ref, out_ref):
    j = pl.program_id(1)
    ma = jax.lax.dot(m1_ref[...], a2_ref[...],
                     preferred_element_type=jnp.float32)
    c = ma * mij_ref[...].astype(jnp.float32)
    chi = c.astype(jnp.bfloat16)
    clo = (c - chi.astype(jnp.float32)).astype(jnp.bfloat16)
    f1 = fv1_ref[...]
    f1hi = f1.astype(jnp.bfloat16)
    f1lo = (f1 - f1hi.astype(jnp.float32)).astype(jnp.bfloat16)
    contrib = (
        jax.lax.dot(chi, f1hi, preferred_element_type=jnp.float32)
        + jax.lax.dot(chi, f1lo, preferred_element_type=jnp.float32)
        + jax.lax.dot(clo, f1hi, preferred_element_type=jnp.float32))

    @pl.when(j == 0)
    def _():
        out_ref[...] = contrib

    @pl.when(j > 0)
    def _():
        out_ref[...] += contrib


_adj = pl.pallas_call(
    _adj_body,
    grid=(NI, NI),
    in_specs=[
        pl.BlockSpec((BI, 128), lambda i, j: (i, 0)),
        pl.BlockSpec((BI, 128), lambda i, j: (i, 0)),
        pl.BlockSpec((8, BI), lambda i, j: (0, j)),
        pl.BlockSpec((8, BI), lambda i, j: (0, j)),
    ],
    out_specs=pl.BlockSpec((BI, BI), lambda i, j: (i, j)),
    out_shape=jax.ShapeDtypeStruct((NP, NP), F8),
)

_fv1 = pl.pallas_call(
    _fv1_body,
    grid=(NI,),
    in_specs=[
        pl.BlockSpec((BI, NP), lambda i: (i, 0)),
        pl.BlockSpec((NP, 128), lambda i: (0, 0)),
        pl.BlockSpec((NP, 128), lambda i: (0, 0)),
        pl.BlockSpec((NP, 128), lambda i: (0, 0)),
        pl.BlockSpec((8, 128), lambda i: (0, 0)),
        pl.BlockSpec((8, 128), lambda i: (0, 0)),
    ],
    out_specs=pl.BlockSpec((BI, D), lambda i: (i, 0)),
    out_shape=jax.ShapeDtypeStruct((NP, D), jnp.float32),
)

_m = pl.pallas_call(
    _m_body,
    grid=(NI, NI),
    in_specs=[
        pl.BlockSpec((BI, NP), lambda i, j: (i, 0)),
        pl.BlockSpec((NP, BI), lambda i, j: (0, j)),
    ],
    out_specs=pl.BlockSpec((BI, BI), lambda i, j: (i, j)),
    out_shape=jax.ShapeDtypeStruct((NP, NP), F8),
)

_fv2 = pl.pallas_call(
    _fv2_body,
    grid=(NI, NI),
    in_specs=[
        pl.BlockSpec((BI, NP), lambda i, j: (i, 0)),
        pl.BlockSpec((NP, BI), lambda i, j: (0, j)),
        pl.BlockSpec((BI, BI), lambda i, j: (i, j)),
        pl.BlockSpec((BI, D), lambda i, j: (j, 0)),
    ],
    out_specs=pl.BlockSpec((BI, D), lambda i, j: (i, 0)),
    out_shape=jax.ShapeDtypeStruct((NP, D), jnp.float32),
)


def kernel(node_locations, time_deadline, W0, b0):
    depot = jax.random.uniform(jax.random.key(1), (1, 2), dtype=jnp.float32)
    loc = jnp.concatenate([depot, node_locations], axis=0)
    tdc = jnp.concatenate(
        [jnp.zeros((1,), jnp.float32), time_deadline[:, 0]], axis=0)
    pad = NP - N_REAL
    x = jnp.concatenate([loc[:, 0], jnp.full((pad,), 1000.0, jnp.float32)])
    y = jnp.concatenate([loc[:, 1], jnp.full((pad,), 2000.0, jnp.float32)])
    t = jnp.concatenate([tdc, jnp.zeros((pad,), jnp.float32)])

    xc = jnp.broadcast_to(x[:, None], (NP, 128))
    yc = jnp.broadcast_to(y[:, None], (NP, 128))
    tc = jnp.broadcast_to(t[:, None], (NP, 128))
    xr = jnp.broadcast_to(x[None, :], (8, NP))
    yr = jnp.broadcast_to(y[None, :], (8, NP))

    wpad = jnp.zeros((8, 128), jnp.float32).at[0:3, :].set(W0.T)
    bpad = jnp.zeros((8, 128), jnp.float32).at[0, :].set(b0)

    a = _adj(xc, yc, xr, yr)
    fv1 = _fv1(a, xc, yc, tc, wpad, bpad)
    m = _m(a, a)
    fv2 = _fv2(m, a, m, fv1)
    return fv2[:N_REAL]


# speedup vs baseline: 1.4096x; 1.0298x over previous
import jax
import jax.numpy as jnp
from jax.experimental import pallas as pl
from jax.experimental.pallas import tpu as pltpu

N_REAL = 2049
NP = 2304
THRESH2 = 0.04 * 0.04
BI = 768
NI = NP // BI
D = 128
F8 = jnp.float8_e4m3fn


def _adjfv1_body(xci_ref, yci_ref, xcj_ref, ycj_ref, dcj_ref,
                 xr_ref, yr_ref, w_ref, b_ref, a_ref, fv1_ref):
    j = pl.program_id(1)
    xi = xci_ref[:, 0:1]
    yi = yci_ref[:, 0:1]
    xj = xr_ref[0:1, :]
    yj = yr_ref[0:1, :]
    dx = xi - xj
    dy = yi - yj
    d2 = dx * dx + dy * dy
    ind = d2 <= THRESH2
    a_ref[...] = ind.astype(F8)

    fv0 = jnp.maximum(
        xcj_ref[:, 0:1] * w_ref[0:1, :]
        + ycj_ref[:, 0:1] * w_ref[1:2, :]
        + dcj_ref[:, 0:1] * w_ref[2:3, :]
        + b_ref[0:1, :], 0.0)
    fhi = fv0.astype(jnp.bfloat16)
    flo = (fv0 - fhi.astype(jnp.float32)).astype(jnp.bfloat16)
    ab = ind.astype(jnp.bfloat16)
    contrib = (
        jax.lax.dot(ab, fhi, preferred_element_type=jnp.float32)
        + jax.lax.dot(ab, flo, preferred_element_type=jnp.float32))

    @pl.when(j == 0)
    def _():
        fv1_ref[...] = contrib

    @pl.when(j > 0)
    def _():
        fv1_ref[...] += contrib


def _m_body(a1_ref, a2_ref, m_ref):
    cnt = jax.lax.dot(a1_ref[...], a2_ref[...],
                      preferred_element_type=jnp.float32)
    m_ref[...] = (cnt > 0.5).astype(F8)


def _fv2_body(m1_ref, a2_ref, mij_ref, f1hi_ref, f1lo_ref, out_ref):
    j = pl.program_id(1)
    ma = jax.lax.dot(m1_ref[...], a2_ref[...],
                     preferred_element_type=jnp.float32)
    c = ma * mij_ref[...].astype(jnp.float32)
    chi = c.astype(jnp.bfloat16)
    clo = (c - chi.astype(jnp.float32)).astype(jnp.bfloat16)
    contrib = (
        jax.lax.dot(chi, f1hi_ref[...], preferred_element_type=jnp.float32)
        + jax.lax.dot(chi, f1lo_ref[...], preferred_element_type=jnp.float32)
        + jax.lax.dot(clo, f1hi_ref[...], preferred_element_type=jnp.float32))

    @pl.when(j == 0)
    def _():
        out_ref[...] = contrib

    @pl.when(j > 0)
    def _():
        out_ref[...] += contrib


_adjfv1 = pl.pallas_call(
    _adjfv1_body,
    grid=(NI, NI),
    in_specs=[
        pl.BlockSpec((BI, 128), lambda i, j: (i, 0)),
        pl.BlockSpec((BI, 128), lambda i, j: (i, 0)),
        pl.BlockSpec((BI, 128), lambda i, j: (j, 0)),
        pl.BlockSpec((BI, 128), lambda i, j: (j, 0)),
        pl.BlockSpec((BI, 128), lambda i, j: (j, 0)),
        pl.BlockSpec((8, BI), lambda i, j: (0, j)),
        pl.BlockSpec((8, BI), lambda i, j: (0, j)),
        pl.BlockSpec((8, 128), lambda i, j: (0, 0)),
        pl.BlockSpec((8, 128), lambda i, j: (0, 0)),
    ],
    out_specs=[
        pl.BlockSpec((BI, BI), lambda i, j: (i, j)),
        pl.BlockSpec((BI, D), lambda i, j: (i, 0)),
    ],
    out_shape=[
        jax.ShapeDtypeStruct((NP, NP), F8),
        jax.ShapeDtypeStruct((NP, D), jnp.float32),
    ],
)

_m = pl.pallas_call(
    _m_body,
    grid=(NI, NI),
    in_specs=[
        pl.BlockSpec((BI, NP), lambda i, j: (i, 0)),
        pl.BlockSpec((NP, BI), lambda i, j: (0, j)),
    ],
    out_specs=pl.BlockSpec((BI, BI), lambda i, j: (i, j)),
    out_shape=jax.ShapeDtypeStruct((NP, NP), F8),
)

_fv2 = pl.pallas_call(
    _fv2_body,
    grid=(NI, NI),
    in_specs=[
        pl.BlockSpec((BI, NP), lambda i, j: (i, 0)),
        pl.BlockSpec((NP, BI), lambda i, j: (0, j)),
        pl.BlockSpec((BI, BI), lambda i, j: (i, j)),
        pl.BlockSpec((BI, D), lambda i, j: (j, 0)),
        pl.BlockSpec((BI, D), lambda i, j: (j, 0)),
    ],
    out_specs=pl.BlockSpec((BI, D), lambda i, j: (i, 0)),
    out_shape=jax.ShapeDtypeStruct((NP, D), jnp.float32),
)


def kernel(node_locations, time_deadline, W0, b0):
    depot = jax.random.uniform(jax.random.key(1), (1, 2), dtype=jnp.float32)
    loc = jnp.concatenate([depot, node_locations], axis=0)
    tdc = jnp.concatenate(
        [jnp.zeros((1,), jnp.float32), time_deadline[:, 0]], axis=0)
    pad = NP - N_REAL
    x = jnp.concatenate([loc[:, 0], jnp.full((pad,), 1000.0, jnp.float32)])
    y = jnp.concatenate([loc[:, 1], jnp.full((pad,), 2000.0, jnp.float32)])
    t = jnp.concatenate([tdc, jnp.zeros((pad,), jnp.float32)])

    xc = jnp.broadcast_to(x[:, None], (NP, 128))
    yc = jnp.broadcast_to(y[:, None], (NP, 128))
    dc = jnp.broadcast_to(t[:, None], (NP, 128))
    xr = jnp.broadcast_to(x[None, :], (8, NP))
    yr = jnp.broadcast_to(y[None, :], (8, NP))

    wpad = jnp.zeros((8, 128), jnp.float32).at[0:3, :].set(W0.T)
    bpad = jnp.zeros((8, 128), jnp.float32).at[0, :].set(b0)

    a, fv1 = _adjfv1(xc, yc, xc, yc, dc, xr, yr, wpad, bpad)
    f1hi = fv1.astype(jnp.bfloat16)
    f1lo = (fv1 - f1hi.astype(jnp.float32)).astype(jnp.bfloat16)
    m = _m(a, a)
    fv2 = _fv2(m, a, m, f1hi, f1lo)
    return fv2[:N_REAL]


# speedup vs baseline: 1.5183x; 1.0771x over previous
import jax
import jax.numpy as jnp
from jax.experimental import pallas as pl
from jax.experimental.pallas import tpu as pltpu

N_REAL = 2049
NP = 2304
THRESH2 = 0.04 * 0.04
BI = 768
NI = NP // BI
D = 128
F8 = jnp.float8_e4m3fn


def _adjfv1_body(xci_ref, yci_ref, xcj_ref, ycj_ref, dcj_ref,
                 xr_ref, yr_ref, w_ref, b_ref, a_ref, fv1_ref):
    j = pl.program_id(1)
    xi = xci_ref[:, 0:1]
    yi = yci_ref[:, 0:1]
    xj = xr_ref[0:1, :]
    yj = yr_ref[0:1, :]
    dx = xi - xj
    dy = yi - yj
    d2 = dx * dx + dy * dy
    ind = d2 <= THRESH2
    a_ref[...] = ind.astype(F8)

    fv0 = jnp.maximum(
        xcj_ref[:, 0:1] * w_ref[0:1, :]
        + ycj_ref[:, 0:1] * w_ref[1:2, :]
        + dcj_ref[:, 0:1] * w_ref[2:3, :]
        + b_ref[0:1, :], 0.0)
    fhi = fv0.astype(jnp.bfloat16)
    flo = (fv0 - fhi.astype(jnp.float32)).astype(jnp.bfloat16)
    ab = ind.astype(jnp.bfloat16)
    contrib = (
        jax.lax.dot(ab, fhi, preferred_element_type=jnp.float32)
        + jax.lax.dot(ab, flo, preferred_element_type=jnp.float32))

    @pl.when(j == 0)
    def _():
        fv1_ref[...] = contrib

    @pl.when(j > 0)
    def _():
        fv1_ref[...] += contrib


def _m_body(a1_ref, a2_ref, m_ref):
    cnt = jax.lax.dot(a1_ref[...], a2_ref[...],
                      preferred_element_type=jnp.float32)
    m_ref[...] = (cnt > 0.5).astype(F8)


def _fv2_body(m1_ref, a2_ref, mij_ref, f1hi_ref, f1lo_ref, out_ref):
    j = pl.program_id(1)
    ma = jax.lax.dot(m1_ref[...], a2_ref[...],
                     preferred_element_type=jnp.float32)
    c = ma * mij_ref[...].astype(jnp.float32)
    chi = c.astype(jnp.bfloat16)
    clo = (c - chi.astype(jnp.float32)).astype(jnp.bfloat16)
    contrib = (
        jax.lax.dot(chi, f1hi_ref[...], preferred_element_type=jnp.float32)
        + jax.lax.dot(chi, f1lo_ref[...], preferred_element_type=jnp.float32)
        + jax.lax.dot(clo, f1hi_ref[...], preferred_element_type=jnp.float32))

    @pl.when(j == 0)
    def _():
        out_ref[...] = contrib

    @pl.when(j > 0)
    def _():
        out_ref[...] += contrib


_adjfv1 = pl.pallas_call(
    _adjfv1_body,
    grid=(NI, NI),
    in_specs=[
        pl.BlockSpec((BI, 128), lambda i, j: (i, 0)),
        pl.BlockSpec((BI, 128), lambda i, j: (i, 0)),
        pl.BlockSpec((BI, 128), lambda i, j: (j, 0)),
        pl.BlockSpec((BI, 128), lambda i, j: (j, 0)),
        pl.BlockSpec((BI, 128), lambda i, j: (j, 0)),
        pl.BlockSpec((8, BI), lambda i, j: (0, j)),
        pl.BlockSpec((8, BI), lambda i, j: (0, j)),
        pl.BlockSpec((8, 128), lambda i, j: (0, 0)),
        pl.BlockSpec((8, 128), lambda i, j: (0, 0)),
    ],
    out_specs=[
        pl.BlockSpec((BI, BI), lambda i, j: (i, j)),
        pl.BlockSpec((BI, D), lambda i, j: (i, 0)),
    ],
    out_shape=[
        jax.ShapeDtypeStruct((NP, NP), F8),
        jax.ShapeDtypeStruct((NP, D), jnp.float32),
    ],
)

BM = 1152
NM = NP // BM

_m = pl.pallas_call(
    _m_body,
    grid=(NM, NM),
    in_specs=[
        pl.BlockSpec((BM, NP), lambda i, j: (i, 0)),
        pl.BlockSpec((NP, BM), lambda i, j: (0, j)),
    ],
    out_specs=pl.BlockSpec((BM, BM), lambda i, j: (i, j)),
    out_shape=jax.ShapeDtypeStruct((NP, NP), F8),
)

_fv2 = pl.pallas_call(
    _fv2_body,
    grid=(NM, NM),
    in_specs=[
        pl.BlockSpec((BM, NP), lambda i, j: (i, 0)),
        pl.BlockSpec((NP, BM), lambda i, j: (0, j)),
        pl.BlockSpec((BM, BM), lambda i, j: (i, j)),
        pl.BlockSpec((BM, D), lambda i, j: (j, 0)),
        pl.BlockSpec((BM, D), lambda i, j: (j, 0)),
    ],
    out_specs=pl.BlockSpec((BM, D), lambda i, j: (i, 0)),
    out_shape=jax.ShapeDtypeStruct((NP, D), jnp.float32),
)


def kernel(node_locations, time_deadline, W0, b0):
    depot = jax.random.uniform(jax.random.key(1), (1, 2), dtype=jnp.float32)
    loc = jnp.concatenate([depot, node_locations], axis=0)
    tdc = jnp.concatenate(
        [jnp.zeros((1,), jnp.float32), time_deadline[:, 0]], axis=0)
    pad = NP - N_REAL
    x = jnp.concatenate([loc[:, 0], jnp.full((pad,), 1000.0, jnp.float32)])
    y = jnp.concatenate([loc[:, 1], jnp.full((pad,), 2000.0, jnp.float32)])
    t = jnp.concatenate([tdc, jnp.zeros((pad,), jnp.float32)])

    xc = jnp.broadcast_to(x[:, None], (NP, 128))
    yc = jnp.broadcast_to(y[:, None], (NP, 128))
    dc = jnp.broadcast_to(t[:, None], (NP, 128))
    xr = jnp.broadcast_to(x[None, :], (8, NP))
    yr = jnp.broadcast_to(y[None, :], (8, NP))

    wpad = jnp.zeros((8, 128), jnp.float32).at[0:3, :].set(W0.T)
    bpad = jnp.zeros((8, 128), jnp.float32).at[0, :].set(b0)

    a, fv1 = _adjfv1(xc, yc, xc, yc, dc, xr, yr, wpad, bpad)
    f1hi = fv1.astype(jnp.bfloat16)
    f1lo = (fv1 - f1hi.astype(jnp.float32)).astype(jnp.bfloat16)
    m = _m(a, a)
    fv2 = _fv2(m, a, m, f1hi, f1lo)
    return fv2[:N_REAL]


# speedup vs baseline: 1.5624x; 1.0291x over previous
import jax
import jax.numpy as jnp
from jax.experimental import pallas as pl
from jax.experimental.pallas import tpu as pltpu

N_REAL = 2049
NP = 2304
THRESH2 = 0.04 * 0.04
BI = 768
NI = NP // BI
D = 128
F8 = jnp.float8_e4m3fn


def _adjfv1_body(xci_ref, yci_ref, xcj_ref, ycj_ref, dcj_ref,
                 xr_ref, yr_ref, w_ref, b_ref, a_ref, fv1_ref):
    j = pl.program_id(1)
    xi = xci_ref[:, 0:1]
    yi = yci_ref[:, 0:1]
    xj = xr_ref[0:1, :]
    yj = yr_ref[0:1, :]
    dx = xi - xj
    dy = yi - yj
    d2 = dx * dx + dy * dy
    ind = d2 <= THRESH2
    a_ref[...] = ind.astype(F8)

    fv0 = jnp.maximum(
        xcj_ref[:, 0:1] * w_ref[0:1, :]
        + ycj_ref[:, 0:1] * w_ref[1:2, :]
        + dcj_ref[:, 0:1] * w_ref[2:3, :]
        + b_ref[0:1, :], 0.0)
    fhi = fv0.astype(jnp.bfloat16)
    flo = (fv0 - fhi.astype(jnp.float32)).astype(jnp.bfloat16)
    ab = ind.astype(jnp.bfloat16)
    contrib = (
        jax.lax.dot(ab, fhi, preferred_element_type=jnp.float32)
        + jax.lax.dot(ab, flo, preferred_element_type=jnp.float32))

    @pl.when(j == 0)
    def _():
        fv1_ref[...] = contrib

    @pl.when(j > 0)
    def _():
        fv1_ref[...] += contrib


def _m_body(a1_ref, a2_ref, m_ref):
    cnt = jax.lax.dot(a1_ref[...], a2_ref[...],
                      preferred_element_type=jnp.float32)
    m_ref[...] = (cnt > 0.5).astype(F8)


def _fv2_body(m1_ref, a2_ref, mij_ref, f1hi_ref, f1lo_ref, out_ref):
    j = pl.program_id(1)
    ma = jax.lax.dot(m1_ref[...], a2_ref[...],
                     preferred_element_type=jnp.float32)
    c = ma * mij_ref[...].astype(jnp.float32)
    chi = c.astype(jnp.bfloat16)
    clo = (c - chi.astype(jnp.float32)).astype(jnp.bfloat16)
    contrib = (
        jax.lax.dot(chi, f1hi_ref[...], preferred_element_type=jnp.float32)
        + jax.lax.dot(chi, f1lo_ref[...], preferred_element_type=jnp.float32)
        + jax.lax.dot(clo, f1hi_ref[...], preferred_element_type=jnp.float32))

    @pl.when(j == 0)
    def _():
        out_ref[...] = contrib

    @pl.when(j > 0)
    def _():
        out_ref[...] += contrib


BA = 1152
NA = NP // BA

_adjfv1 = pl.pallas_call(
    _adjfv1_body,
    grid=(NA, NA),
    in_specs=[
        pl.BlockSpec((BA, 128), lambda i, j: (i, 0)),
        pl.BlockSpec((BA, 128), lambda i, j: (i, 0)),
        pl.BlockSpec((BA, 128), lambda i, j: (j, 0)),
        pl.BlockSpec((BA, 128), lambda i, j: (j, 0)),
        pl.BlockSpec((BA, 128), lambda i, j: (j, 0)),
        pl.BlockSpec((8, BA), lambda i, j: (0, j)),
        pl.BlockSpec((8, BA), lambda i, j: (0, j)),
        pl.BlockSpec((8, 128), lambda i, j: (0, 0)),
        pl.BlockSpec((8, 128), lambda i, j: (0, 0)),
    ],
    out_specs=[
        pl.BlockSpec((BA, BA), lambda i, j: (i, j)),
        pl.BlockSpec((BA, D), lambda i, j: (i, 0)),
    ],
    out_shape=[
        jax.ShapeDtypeStruct((NP, NP), F8),
        jax.ShapeDtypeStruct((NP, D), jnp.float32),
    ],
    compiler_params=pltpu.CompilerParams(
        dimension_semantics=("parallel", "arbitrary")),
)

BM = 1152
NM = NP // BM

_m = pl.pallas_call(
    _m_body,
    grid=(NM, NM),
    in_specs=[
        pl.BlockSpec((BM, NP), lambda i, j: (i, 0)),
        pl.BlockSpec((NP, BM), lambda i, j: (0, j)),
    ],
    out_specs=pl.BlockSpec((BM, BM), lambda i, j: (i, j)),
    out_shape=jax.ShapeDtypeStruct((NP, NP), F8),
)

_fv2 = pl.pallas_call(
    _fv2_body,
    grid=(NM, NM),
    in_specs=[
        pl.BlockSpec((BM, NP), lambda i, j: (i, 0)),
        pl.BlockSpec((NP, BM), lambda i, j: (0, j)),
        pl.BlockSpec((BM, BM), lambda i, j: (i, j)),
        pl.BlockSpec((BM, D), lambda i, j: (j, 0)),
        pl.BlockSpec((BM, D), lambda i, j: (j, 0)),
    ],
    out_specs=pl.BlockSpec((BM, D), lambda i, j: (i, 0)),
    out_shape=jax.ShapeDtypeStruct((NP, D), jnp.float32),
)


def kernel(node_locations, time_deadline, W0, b0):
    depot = jax.random.uniform(jax.random.key(1), (1, 2), dtype=jnp.float32)
    loc = jnp.concatenate([depot, node_locations], axis=0)
    tdc = jnp.concatenate(
        [jnp.zeros((1,), jnp.float32), time_deadline[:, 0]], axis=0)
    pad = NP - N_REAL
    x = jnp.concatenate([loc[:, 0], jnp.full((pad,), 1000.0, jnp.float32)])
    y = jnp.concatenate([loc[:, 1], jnp.full((pad,), 2000.0, jnp.float32)])
    t = jnp.concatenate([tdc, jnp.zeros((pad,), jnp.float32)])

    xc = jnp.broadcast_to(x[:, None], (NP, 128))
    yc = jnp.broadcast_to(y[:, None], (NP, 128))
    dc = jnp.broadcast_to(t[:, None], (NP, 128))
    xr = jnp.broadcast_to(x[None, :], (8, NP))
    yr = jnp.broadcast_to(y[None, :], (8, NP))

    wpad = jnp.zeros((8, 128), jnp.float32).at[0:3, :].set(W0.T)
    bpad = jnp.zeros((8, 128), jnp.float32).at[0, :].set(b0)

    a, fv1 = _adjfv1(xc, yc, xc, yc, dc, xr, yr, wpad, bpad)
    f1hi = fv1.astype(jnp.bfloat16)
    f1lo = (fv1 - f1hi.astype(jnp.float32)).astype(jnp.bfloat16)
    m = _m(a, a)
    fv2 = _fv2(m, a, m, f1hi, f1lo)
    return fv2[:N_REAL]


# speedup vs baseline: 1.5964x; 1.0218x over previous
import jax
import jax.numpy as jnp
from jax.experimental import pallas as pl
from jax.experimental.pallas import tpu as pltpu

N_REAL = 2049
NP = 2304
THRESH2 = 0.04 * 0.04
BI = 768
NI = NP // BI
D = 128
F8 = jnp.float8_e4m3fn


def _adjfv1_body(xci_ref, yci_ref, xcj_ref, ycj_ref, dcj_ref,
                 xr_ref, yr_ref, w_ref, b_ref, a_ref, fv1_ref):
    j = pl.program_id(1)
    xi = xci_ref[:, 0:1]
    yi = yci_ref[:, 0:1]
    xj = xr_ref[0:1, :]
    yj = yr_ref[0:1, :]
    dx = xi - xj
    dy = yi - yj
    d2 = dx * dx + dy * dy
    ind = d2 <= THRESH2
    a_ref[...] = ind.astype(F8)

    fv0 = jnp.maximum(
        xcj_ref[:, 0:1] * w_ref[0:1, :]
        + ycj_ref[:, 0:1] * w_ref[1:2, :]
        + dcj_ref[:, 0:1] * w_ref[2:3, :]
        + b_ref[0:1, :], 0.0)
    fhi = fv0.astype(jnp.bfloat16)
    flo = (fv0 - fhi.astype(jnp.float32)).astype(jnp.bfloat16)
    ab = ind.astype(jnp.bfloat16)
    contrib = (
        jax.lax.dot(ab, fhi, preferred_element_type=jnp.float32)
        + jax.lax.dot(ab, flo, preferred_element_type=jnp.float32))

    @pl.when(j == 0)
    def _():
        fv1_ref[...] = contrib

    @pl.when(j > 0)
    def _():
        fv1_ref[...] += contrib


KC = 2176


def _m_body(a1_ref, a2_ref, m_ref):
    cnt = jax.lax.dot(a1_ref[:, :KC], a2_ref[:KC, :],
                      preferred_element_type=jnp.float32)
    m_ref[...] = (cnt > 0.5).astype(F8)


def _fv2_body(m1_ref, a2_ref, mij_ref, f1hi_ref, f1lo_ref, out_ref):
    j = pl.program_id(1)
    ma = jax.lax.dot(m1_ref[:, :KC], a2_ref[:KC, :],
                     preferred_element_type=jnp.float32)
    c = ma * mij_ref[...].astype(jnp.float32)
    chi = c.astype(jnp.bfloat16)
    clo = (c - chi.astype(jnp.float32)).astype(jnp.bfloat16)
    contrib = (
        jax.lax.dot(chi, f1hi_ref[...], preferred_element_type=jnp.float32)
        + jax.lax.dot(chi, f1lo_ref[...], preferred_element_type=jnp.float32)
        + jax.lax.dot(clo, f1hi_ref[...], preferred_element_type=jnp.float32))

    @pl.when(j == 0)
    def _():
        out_ref[...] = contrib

    @pl.when(j > 0)
    def _():
        out_ref[...] += contrib


BA = 1152
NA = NP // BA

_adjfv1 = pl.pallas_call(
    _adjfv1_body,
    grid=(NA, NA),
    in_specs=[
        pl.BlockSpec((BA, 128), lambda i, j: (i, 0)),
        pl.BlockSpec((BA, 128), lambda i, j: (i, 0)),
        pl.BlockSpec((BA, 128), lambda i, j: (j, 0)),
        pl.BlockSpec((BA, 128), lambda i, j: (j, 0)),
        pl.BlockSpec((BA, 128), lambda i, j: (j, 0)),
        pl.BlockSpec((8, BA), lambda i, j: (0, j)),
        pl.BlockSpec((8, BA), lambda i, j: (0, j)),
        pl.BlockSpec((8, 128), lambda i, j: (0, 0)),
        pl.BlockSpec((8, 128), lambda i, j: (0, 0)),
    ],
    out_specs=[
        pl.BlockSpec((BA, BA), lambda i, j: (i, j)),
        pl.BlockSpec((BA, D), lambda i, j: (i, 0)),
    ],
    out_shape=[
        jax.ShapeDtypeStruct((NP, NP), F8),
        jax.ShapeDtypeStruct((NP, D), jnp.float32),
    ],
    compiler_params=pltpu.CompilerParams(
        dimension_semantics=("parallel", "arbitrary")),
)

BM = 1152
NM = NP // BM

_m = pl.pallas_call(
    _m_body,
    grid=(NM, NM),
    in_specs=[
        pl.BlockSpec((BM, NP), lambda i, j: (i, 0)),
        pl.BlockSpec((NP, BM), lambda i, j: (0, j)),
    ],
    out_specs=pl.BlockSpec((BM, BM), lambda i, j: (i, j)),
    out_shape=jax.ShapeDtypeStruct((NP, NP), F8),
)

_fv2 = pl.pallas_call(
    _fv2_body,
    grid=(NM, NM),
    in_specs=[
        pl.BlockSpec((BM, NP), lambda i, j: (i, 0)),
        pl.BlockSpec((NP, BM), lambda i, j: (0, j)),
        pl.BlockSpec((BM, BM), lambda i, j: (i, j)),
        pl.BlockSpec((BM, D), lambda i, j: (j, 0)),
        pl.BlockSpec((BM, D), lambda i, j: (j, 0)),
    ],
    out_specs=pl.BlockSpec((BM, D), lambda i, j: (i, 0)),
    out_shape=jax.ShapeDtypeStruct((NP, D), jnp.float32),
)


def kernel(node_locations, time_deadline, W0, b0):
    depot = jax.random.uniform(jax.random.key(1), (1, 2), dtype=jnp.float32)
    loc = jnp.concatenate([depot, node_locations], axis=0)
    tdc = jnp.concatenate(
        [jnp.zeros((1,), jnp.float32), time_deadline[:, 0]], axis=0)
    pad = NP - N_REAL
    x = jnp.concatenate([loc[:, 0], jnp.full((pad,), 1000.0, jnp.float32)])
    y = jnp.concatenate([loc[:, 1], jnp.full((pad,), 2000.0, jnp.float32)])
    t = jnp.concatenate([tdc, jnp.zeros((pad,), jnp.float32)])

    xc = jnp.broadcast_to(x[:, None], (NP, 128))
    yc = jnp.broadcast_to(y[:, None], (NP, 128))
    dc = jnp.broadcast_to(t[:, None], (NP, 128))
    xr = jnp.broadcast_to(x[None, :], (8, NP))
    yr = jnp.broadcast_to(y[None, :], (8, NP))

    wpad = jnp.zeros((8, 128), jnp.float32).at[0:3, :].set(W0.T)
    bpad = jnp.zeros((8, 128), jnp.float32).at[0, :].set(b0)

    a, fv1 = _adjfv1(xc, yc, xc, yc, dc, xr, yr, wpad, bpad)
    f1hi = fv1.astype(jnp.bfloat16)
    f1lo = (fv1 - f1hi.astype(jnp.float32)).astype(jnp.bfloat16)
    m = _m(a, a)
    fv2 = _fv2(m, a, m, f1hi, f1lo)
    return fv2[:N_REAL]
